# baseline (speedup 1.0000x reference)
import functools

import jax
import jax.numpy as jnp
from jax import lax
from jax.experimental import pallas as pl
from jax.experimental.pallas import tpu as pltpu
from jax.experimental.pallas import tpu_sc as plsc

N = 10000
NP = 10240
E = 320000
C = 128
NC = 2
NS = 16
NW = NC * NS
BG = 128
CPT = 80
NCHUNK = NW * CPT
EPAD = NCHUNK * BG
BE = 2048
BN = 1024
EPS = 1e-5


def _mesh():
    return plsc.VectorSubcoreMesh(core_axis_name="c", subcore_axis_name="s")


KG = 3
NGRP = CPT // KG
NLEFT = CPT - NGRP * KG


def _sc_gather(h_pad, idx_d2, idx_s2):
    @functools.partial(
        pl.kernel,
        mesh=_mesh(),
        out_type=[jax.ShapeDtypeStruct((EPAD, C), jnp.float32),
                  jax.ShapeDtypeStruct((EPAD, C), jnp.float32)],
        scratch_types=[pltpu.VMEM((CPT, BG), jnp.int32),
                       pltpu.VMEM((CPT, BG), jnp.int32),
                       pltpu.VMEM((KG * BG, C), jnp.float32),
                       pltpu.VMEM((KG * BG, C), jnp.float32),
                       pltpu.SemaphoreType.DMA,
                       pltpu.SemaphoreType.DMA],
    )
    def k(h_ref, id_ref, is_ref, gd_ref, gs_ref, ivd, ivs, rd, rs, gsem, ssem):
        wid = lax.axis_index("s") * NC + lax.axis_index("c")
        cb0 = wid * CPT
        pltpu.sync_copy(id_ref.at[pl.ds(cb0, CPT)], ivd)
        pltpu.sync_copy(is_ref.at[pl.ds(cb0, CPT)], ivs)

        def group(g, _):
            cb = cb0 + g * KG
            ds = []
            for t in range(KG):
                ds.append(pltpu.async_copy(
                    h_ref.at[ivd.at[g * KG + t]], rd.at[pl.ds(t * BG, BG)], gsem))
                ds.append(pltpu.async_copy(
                    h_ref.at[ivs.at[g * KG + t]], rs.at[pl.ds(t * BG, BG)], gsem))
            for c in ds:
                c.wait()
            pltpu.async_copy(rd, gd_ref.at[pl.ds(cb * BG, KG * BG)], ssem).wait()
            pltpu.async_copy(rs, gs_ref.at[pl.ds(cb * BG, KG * BG)], ssem).wait()
            return 0

        lax.fori_loop(0, NGRP, group, 0)

        for r in range(NLEFT):
            j = NGRP * KG + r
            c1 = pltpu.async_copy(h_ref.at[ivd.at[j]], rd.at[pl.ds(0, BG)], gsem)
            c2 = pltpu.async_copy(h_ref.at[ivs.at[j]], rs.at[pl.ds(0, BG)], gsem)
            c1.wait()
            c2.wait()
            cb = cb0 + j
            pltpu.sync_copy(rd.at[pl.ds(0, BG)], gd_ref.at[pl.ds(cb * BG, BG)])
            pltpu.sync_copy(rs.at[pl.ds(0, BG)], gs_ref.at[pl.ds(cb * BG, BG)])

    return k(h_pad, idx_d2, idx_s2)


def _sc_scatter(msg, idx_d2, zeros_np):
    @functools.partial(
        pl.kernel,
        mesh=_mesh(),
        out_type=jax.ShapeDtypeStruct((NC, NP, C), jnp.float32),
        scratch_types=[pltpu.VMEM((CPT, BG), jnp.int32),
                       pltpu.VMEM((BG, C), jnp.float32),
                       pltpu.VMEM((BG, C), jnp.float32),
                       pltpu.VMEM_SHARED((NP, C), jnp.float32),
                       pltpu.SemaphoreType.DMA,
                       pltpu.SemaphoreType.DMA],
    )
    def k(m_ref, id_ref, z_ref, out_ref, iv, mva, mvb, acc, sa, sb):
        cid = lax.axis_index("c")
        sid = lax.axis_index("s")
        wid = sid * NC + cid
        rpt = NP // NS
        pltpu.sync_copy(z_ref.at[pl.ds(sid * rpt, rpt)],
                        acc.at[pl.ds(sid * rpt, rpt)])
        cb0 = wid * CPT
        pltpu.sync_copy(id_ref.at[pl.ds(cb0, CPT)], iv)
        plsc.subcore_barrier()
        pltpu.async_copy(m_ref.at[pl.ds(cb0 * BG, BG)], mva, sa)

        def group(g, _):
            cb = cb0 + g

            @pl.when(g % 2 == 0)
            def _():
                @pl.when(g + 1 < CPT)
                def _():
                    pltpu.async_copy(m_ref.at[pl.ds((cb + 1) * BG, BG)], mvb, sb)
                pltpu.make_async_copy(m_ref.at[pl.ds(cb * BG, BG)],
                                      mva, sa).wait()
                pltpu.sync_copy(mva, acc.at[iv.at[g]], add=True)

            @pl.when(g % 2 == 1)
            def _():
                @pl.when(g + 1 < CPT)
                def _():
                    pltpu.async_copy(m_ref.at[pl.ds((cb + 1) * BG, BG)], mva, sa)
                pltpu.make_async_copy(m_ref.at[pl.ds(cb * BG, BG)],
                                      mvb, sb).wait()
                pltpu.sync_copy(mvb, acc.at[iv.at[g]], add=True)

            return 0

        lax.fori_loop(0, CPT, group, 0)
        plsc.subcore_barrier()
        pltpu.sync_copy(acc.at[pl.ds(sid * rpt, rpt)],
                        out_ref.at[cid, pl.ds(sid * rpt, rpt)])

    return k(msg, idx_d2, zeros_np)


def _sc_counts(idx_d2, zeros_np, ones_e):
    cp = _sc_scatter(ones_e, idx_d2, zeros_np)
    return (cp[0] + cp[1])[:, :16]


def _k_pre(x0_ref, x1_ref, am_ref, wa_ref, wv_ref, ba_ref, bv_ref,
           y_ref, st_ref, acc):
    i = pl.program_id(0)
    ya = jnp.dot(x0_ref[...], wa_ref[...], preferred_element_type=jnp.float32) + ba_ref[...]
    yv = jnp.dot(x1_ref[...], wv_ref[...], preferred_element_type=jnp.float32) + bv_ref[...]
    m = jnp.broadcast_to(am_ref[...][:, 0:1], (BN, C))
    y = m * ya + (1.0 - m) * yv
    y_ref[...] = y
    row = i * BN + lax.broadcasted_iota(jnp.int32, (BN, C), 0)
    rm = jnp.where(row < N, 1.0, 0.0)

    @pl.when(i == 0)
    def _():
        acc[...] = jnp.zeros_like(acc)

    ym = y * rm
    acc[0:1, :] += jnp.sum(ym, axis=0, keepdims=True)
    acc[1:2, :] += jnp.sum(ym * y, axis=0, keepdims=True)

    @pl.when(i == pl.num_programs(0) - 1)
    def _():
        st_ref[...] = acc[...]


def _tc_pre(x0p, x1p, am8, W0a, W0v, b0a, b0v):
    grid = NP // BN
    return pl.pallas_call(
        _k_pre,
        grid=(grid,),
        in_specs=[pl.BlockSpec((BN, C), lambda i: (i, 0)),
                  pl.BlockSpec((BN, C), lambda i: (i, 0)),
                  pl.BlockSpec((BN, 8), lambda i: (i, 0)),
                  pl.BlockSpec((C, C), lambda i: (0, 0)),
                  pl.BlockSpec((C, C), lambda i: (0, 0)),
                  pl.BlockSpec((1, C), lambda i: (0, 0)),
                  pl.BlockSpec((1, C), lambda i: (0, 0))],
        out_specs=[pl.BlockSpec((BN, C), lambda i: (i, 0)),
                   pl.BlockSpec((8, C), lambda i: (0, 0))],
        out_shape=[jax.ShapeDtypeStruct((NP, C), jnp.float32),
                   jax.ShapeDtypeStruct((8, C), jnp.float32)],
        scratch_shapes=[pltpu.VMEM((8, C), jnp.float32)],
    )(x0p, x1p, am8, W0a, W0v, b0a, b0v)


def _k_statsA(gd_ref, gs_ref, st_ref, acc):
    i = pl.program_id(0)

    @pl.when(i == 0)
    def _():
        acc[...] = jnp.zeros_like(acc)

    xi = gd_ref[...]
    xd = gs_ref[...] - xi
    acc[0:1, :] += jnp.sum(xi, axis=0, keepdims=True)
    acc[1:2, :] += jnp.sum(xi * xi, axis=0, keepdims=True)
    acc[2:3, :] += jnp.sum(xd, axis=0, keepdims=True)
    acc[3:4, :] += jnp.sum(xd * xd, axis=0, keepdims=True)

    @pl.when(i == pl.num_programs(0) - 1)
    def _():
        st_ref[...] = acc[...]


def _tc_statsA(gd, gs):
    return pl.pallas_call(
        _k_statsA,
        grid=(EPAD // BE,),
        in_specs=[pl.BlockSpec((BE, C), lambda i: (i, 0)),
                  pl.BlockSpec((BE, C), lambda i: (i, 0))],
        out_specs=pl.BlockSpec((8, C), lambda i: (0, 0)),
        out_shape=jax.ShapeDtypeStruct((8, C), jnp.float32),
        scratch_shapes=[pltpu.VMEM((8, C), jnp.float32)],
    )(gd, gs)


def _k_passB(gd_ref, gs_ref, w8_ref, sa_ref, ha_ref, sb_ref, hb_ref,
             w1a_ref, w1b_ref, h1_ref, st_ref, acc):
    i = pl.program_id(0)

    @pl.when(i == 0)
    def _():
        acc[...] = jnp.zeros_like(acc)

    xi = gd_ref[...]
    xd = gs_ref[...] - xi
    ra = jnp.maximum(xi * sa_ref[...] + ha_ref[...], 0.0)
    rb = jnp.maximum(xd * sb_ref[...] + hb_ref[...], 0.0)
    h1 = (jnp.dot(ra, w1a_ref[...], preferred_element_type=jnp.float32)
          + jnp.dot(rb, w1b_ref[...], preferred_element_type=jnp.float32))
    h1_ref[...] = h1
    m = jnp.broadcast_to(w8_ref[...][:, 0:1], (BE, C))
    hm = h1 * m
    acc[0:1, :] += jnp.sum(hm, axis=0, keepdims=True)
    acc[1:2, :] += jnp.sum(hm * h1, axis=0, keepdims=True)

    @pl.when(i == pl.num_programs(0) - 1)
    def _():
        st_ref[...] = acc[...]


def _tc_passB(gd, gs, w8, sca, sha, scb, shb, W1a, W1b):
    return pl.pallas_call(
        _k_passB,
        grid=(EPAD // BE,),
        in_specs=[pl.BlockSpec((BE, C), lambda i: (i, 0)),
                  pl.BlockSpec((BE, C), lambda i: (i, 0)),
                  pl.BlockSpec((BE, 8), lambda i: (i, 0)),
                  pl.BlockSpec((1, C), lambda i: (0, 0)),
                  pl.BlockSpec((1, C), lambda i: (0, 0)),
                  pl.BlockSpec((1, C), lambda i: (0, 0)),
                  pl.BlockSpec((1, C), lambda i: (0, 0)),
                  pl.BlockSpec((C, C), lambda i: (0, 0)),
                  pl.BlockSpec((C, C), lambda i: (0, 0))],
        out_specs=[pl.BlockSpec((BE, C), lambda i: (i, 0)),
                   pl.BlockSpec((8, C), lambda i: (0, 0))],
        out_shape=[jax.ShapeDtypeStruct((EPAD, C), jnp.float32),
                   jax.ShapeDtypeStruct((8, C), jnp.float32)],
        scratch_shapes=[pltpu.VMEM((8, C), jnp.float32)],
    )(gd, gs, w8, sca, sha, scb, shb, W1a, W1b)


def _k_passC(h1_ref, sc_ref, sh_ref, w2_ref, msg_ref):
    r = jnp.maximum(h1_ref[...] * sc_ref[...] + sh_ref[...], 0.0)
    msg_ref[...] = jnp.dot(r, w2_ref[...], preferred_element_type=jnp.float32)


def _tc_passC(h1, sc2, sh2, W2):
    return pl.pallas_call(
        _k_passC,
        grid=(EPAD // BE,),
        in_specs=[pl.BlockSpec((BE, C), lambda i: (i, 0)),
                  pl.BlockSpec((1, C), lambda i: (0, 0)),
                  pl.BlockSpec((1, C), lambda i: (0, 0)),
                  pl.BlockSpec((C, C), lambda i: (0, 0))],
        out_specs=pl.BlockSpec((BE, C), lambda i: (i, 0)),
        out_shape=jax.ShapeDtypeStruct((EPAD, C), jnp.float32),
    )(h1, sc2, sh2, W2)


def _k_combine(a0_ref, a1_ref, ct_ref, res_ref, y_ref, st_ref, acc):
    i = pl.program_id(0)
    c = jnp.broadcast_to(ct_ref[...][:, 0:1], (BN, C))
    inv = 1.0 / jnp.maximum(c, 1.0)
    row = i * BN + lax.broadcasted_iota(jnp.int32, (BN, C), 0)
    rm = jnp.where(row < N, 1.0, 0.0)
    y = ((a0_ref[0] + a1_ref[0]) * inv + res_ref[...]) * rm
    y_ref[...] = y

    @pl.when(i == 0)
    def _():
        acc[...] = jnp.zeros_like(acc)

    acc[0:1, :] += jnp.sum(y, axis=0, keepdims=True)
    acc[1:2, :] += jnp.sum(y * y, axis=0, keepdims=True)

    @pl.when(i == pl.num_programs(0) - 1)
    def _():
        st_ref[...] = acc[...]


def _tc_combine(acc2, cnt16, res):
    return pl.pallas_call(
        _k_combine,
        grid=(NP // BN,),
        in_specs=[pl.BlockSpec((1, BN, C), lambda i: (0, i, 0)),
                  pl.BlockSpec((1, BN, C), lambda i: (1, i, 0)),
                  pl.BlockSpec((BN, 16), lambda i: (i, 0)),
                  pl.BlockSpec((BN, C), lambda i: (i, 0))],
        out_specs=[pl.BlockSpec((BN, C), lambda i: (i, 0)),
                   pl.BlockSpec((8, C), lambda i: (0, 0))],
        out_shape=[jax.ShapeDtypeStruct((NP, C), jnp.float32),
                   jax.ShapeDtypeStruct((8, C), jnp.float32)],
        scratch_shapes=[pltpu.VMEM((8, C), jnp.float32)],
    )(acc2, acc2, cnt16, res)


def _k_norm(y_ref, sc_ref, sh_ref, h_ref):
    i = pl.program_id(0)
    row = i * BN + lax.broadcasted_iota(jnp.int32, (BN, C), 0)
    rm = jnp.where(row < N, 1.0, 0.0)
    h_ref[...] = jnp.maximum(y_ref[...] * sc_ref[...] + sh_ref[...], 0.0) * rm


def _tc_norm(y, sc, sh):
    return pl.pallas_call(
        _k_norm,
        grid=(NP // BN,),
        in_specs=[pl.BlockSpec((BN, C), lambda i: (i, 0)),
                  pl.BlockSpec((1, C), lambda i: (0, 0)),
                  pl.BlockSpec((1, C), lambda i: (0, 0))],
        out_specs=pl.BlockSpec((BN, C), lambda i: (i, 0)),
        out_shape=jax.ShapeDtypeStruct((NP, C), jnp.float32),
    )(y, sc, sh)


def _k_final(a0_ref, a1_ref, ct_ref, res_ref, fw_ref, fb_ref, o_ref):
    c = jnp.broadcast_to(ct_ref[...][:, 0:1], (1000, C))
    inv = 1.0 / jnp.maximum(c, 1.0)
    y = (a0_ref[0] + a1_ref[0]) * inv + res_ref[...]
    o_ref[...] = jnp.dot(y, fw_ref[...], preferred_element_type=jnp.float32) + fb_ref[...]


def _tc_final(acc2, cnt16, res, fcW, fcb):
    return pl.pallas_call(
        _k_final,
        grid=(10,),
        in_specs=[pl.BlockSpec((1, 1000, C), lambda i: (0, i, 0)),
                  pl.BlockSpec((1, 1000, C), lambda i: (1, i, 0)),
                  pl.BlockSpec((1000, 16), lambda i: (i, 0)),
                  pl.BlockSpec((1000, C), lambda i: (i, 0)),
                  pl.BlockSpec((C, 2), lambda i: (0, 0)),
                  pl.BlockSpec((1, 2), lambda i: (0, 0))],
        out_specs=pl.BlockSpec((1000, 2), lambda i: (i, 0)),
        out_shape=jax.ShapeDtypeStruct((N, 2), jnp.float32),
    )(acc2, acc2, cnt16, res, fcW, fcb)


def _fold(s_sum, s_sq, cnt, g, b):
    m = s_sum / cnt
    v = s_sq / cnt - m * m
    rstd = g / jnp.sqrt(v + EPS)
    return rstd.reshape(1, C), (b - m * rstd).reshape(1, C)


def _conv(h, idx_s, idx_d, w8, cnt, bn1g, bn1b, W1, bn2g, bn2b, W2, zeros_np):
    gd, gs = _sc_gather(h, idx_d, idx_s)
    st1 = _tc_statsA(gd, gs)
    sca, sha = _fold(st1[0], st1[1], cnt, bn1g[:C], bn1b[:C])
    scb, shb = _fold(st1[2], st1[3], cnt, bn1g[C:], bn1b[C:])
    h1, st2 = _tc_passB(gd, gs, w8, sca, sha, scb, shb, W1[:C], W1[C:])
    sc2, sh2 = _fold(st2[0], st2[1], cnt, bn2g, bn2b)
    msg = _tc_passC(h1, sc2, sh2, W2)
    return _sc_scatter(msg, idx_d, zeros_np)


def kernel(x, edge_index, edge_delta, edge_self, audio_node_mask,
           W0a, b0a, W0v, b0v, g0, be0,
           l1_bn1g, l1_bn1b, l1_W1, l1_bn2g, l1_bn2b, l1_W2,
           l2_bn1g, l2_bn1b, l2_W1, l2_bn2g, l2_bn2b, l2_W2,
           l3_bn1g, l3_bn1b, l3_W1, l3_bn2g, l3_bn2b, l3_W2,
           l4_bn1g, l4_bn1b, l4_W1, l4_bn2g, l4_bn2b, l4_W2,
           g1, be1, g2, be2, g3, be3, fcW, fcb):
    f32 = jnp.float32
    src = edge_index[0]
    dst = edge_index[1]
    m1 = edge_delta < 1
    m2 = ((edge_delta >= 1) & (edge_delta < 3)) | (edge_self == 1)

    def prep(m):
        w = m.astype(f32)
        s = jnp.pad(jnp.where(m, src, N), (0, EPAD - E),
                    constant_values=N).astype(jnp.int32).reshape(NCHUNK, BG)
        d = jnp.pad(jnp.where(m, dst, N), (0, EPAD - E),
                    constant_values=N).astype(jnp.int32).reshape(NCHUNK, BG)
        w8 = jnp.pad(jnp.tile(w[:, None], (1, 8)), [(0, EPAD - E), (0, 0)])
        return s, d, w8

    s1i, d1i, w8_1 = prep(m1)
    s2i, d2i, w8_2 = prep(m2)

    zeros_np = jnp.zeros((NP, C), f32)
    ones_e = jnp.ones((EPAD, C), f32)

    cnt16_1 = _sc_counts(d1i, zeros_np, ones_e)
    cnt16_2 = _sc_counts(d2i, zeros_np, ones_e)
    cnt1 = jnp.sum(cnt16_1[:N, 0])
    cnt2 = jnp.sum(cnt16_2[:N, 0])
    cnt1 = jnp.maximum(cnt1, 1.0)
    cnt2 = jnp.maximum(cnt2, 1.0)

    pad_n = [(0, NP - N), (0, 0)]
    x0p = jnp.pad(x[:, 0, :], pad_n)
    x1p = jnp.pad(x[:, 1, :], pad_n)
    am8 = jnp.pad(jnp.tile(audio_node_mask.astype(f32)[:, None], (1, 8)),
                  [(0, NP - N), (0, 0)])

    y0, st0 = _tc_pre(x0p, x1p, am8, W0a, W0v,
                      b0a.reshape(1, C), b0v.reshape(1, C))
    sc0, sh0 = _fold(st0[0], st0[1], float(N), g0, be0)
    gf = _tc_norm(y0, sc0, sh0)

    params = [(l1_bn1g, l1_bn1b, l1_W1, l1_bn2g, l1_bn2b, l1_W2),
              (l2_bn1g, l2_bn1b, l2_W1, l2_bn2g, l2_bn2b, l2_W2),
              (l3_bn1g, l3_bn1b, l3_W1, l3_bn2g, l3_bn2b, l3_W2),
              (l4_bn1g, l4_bn1b, l4_W1, l4_bn2g, l4_bn2b, l4_W2)]
    gbs = [(g1, be1), (g2, be2), (g3, be3)]

    h = gf
    for l in range(4):
        p = params[l]
        acc_a = _conv(h, s1i, d1i, w8_1, cnt1, *p, zeros_np)
        a, _ = _tc_combine(acc_a, cnt16_1, zeros_np)
        acc_b = _conv(a, s2i, d2i, w8_2, cnt2, *p, zeros_np)
        res = zeros_np if l == 0 else h
        if l < 3:
            y, stn = _tc_combine(acc_b, cnt16_2, res)
            scn, shn = _fold(stn[0], stn[1], float(N), gbs[l][0], gbs[l][1])
            h = _tc_norm(y, scn, shn)
        else:
            return _tc_final(acc_b, cnt16_2, res, fcW, fcb.reshape(1, 2))


# speedup vs baseline: 14.5964x; 14.5964x over previous
import functools

import jax
import jax.numpy as jnp
from jax import lax
from jax.experimental import pallas as pl
from jax.experimental.pallas import tpu as pltpu
from jax.experimental.pallas import tpu_sc as plsc

N = 10000
NP = 10240
E = 320000
C = 128
NC = 2
NS = 16
NW = NC * NS
BG = 128
CPT = 80
NCHUNK = NW * CPT
EPAD = NCHUNK * BG
BE = 2048
BN = 1024
EPS = 1e-5


def _mesh():
    return plsc.VectorSubcoreMesh(core_axis_name="c", subcore_axis_name="s")


KG = 3
NGRP = CPT // KG
NLEFT = CPT - NGRP * KG


def _sc_gather(h_pad, idx_d2, idx_s2):
    @functools.partial(
        pl.kernel,
        mesh=_mesh(),
        out_type=[jax.ShapeDtypeStruct((EPAD, C), jnp.float32),
                  jax.ShapeDtypeStruct((EPAD, C), jnp.float32)],
        scratch_types=[pltpu.VMEM((CPT, BG), jnp.int32),
                       pltpu.VMEM((CPT, BG), jnp.int32),
                       pltpu.VMEM((KG * BG, C), jnp.float32),
                       pltpu.VMEM((KG * BG, C), jnp.float32),
                       pltpu.SemaphoreType.DMA,
                       pltpu.SemaphoreType.DMA],
    )
    def k(h_ref, id_ref, is_ref, gd_ref, gs_ref, ivd, ivs, rd, rs, gsem, ssem):
        wid = lax.axis_index("s") * NC + lax.axis_index("c")
        cb0 = wid * CPT
        pltpu.sync_copy(id_ref.at[pl.ds(cb0, CPT)], ivd)
        pltpu.sync_copy(is_ref.at[pl.ds(cb0, CPT)], ivs)

        def group(g, _):
            cb = cb0 + g * KG
            ds = []
            for t in range(KG):
                ds.append(pltpu.async_copy(
                    h_ref.at[ivd.at[g * KG + t]], rd.at[pl.ds(t * BG, BG)], gsem))
                ds.append(pltpu.async_copy(
                    h_ref.at[ivs.at[g * KG + t]], rs.at[pl.ds(t * BG, BG)], gsem))
            for c in ds:
                c.wait()
            pltpu.async_copy(rd, gd_ref.at[pl.ds(cb * BG, KG * BG)], ssem).wait()
            pltpu.async_copy(rs, gs_ref.at[pl.ds(cb * BG, KG * BG)], ssem).wait()
            return 0

        lax.fori_loop(0, NGRP, group, 0)

        for r in range(NLEFT):
            j = NGRP * KG + r
            c1 = pltpu.async_copy(h_ref.at[ivd.at[j]], rd.at[pl.ds(0, BG)], gsem)
            c2 = pltpu.async_copy(h_ref.at[ivs.at[j]], rs.at[pl.ds(0, BG)], gsem)
            c1.wait()
            c2.wait()
            cb = cb0 + j
            pltpu.sync_copy(rd.at[pl.ds(0, BG)], gd_ref.at[pl.ds(cb * BG, BG)])
            pltpu.sync_copy(rs.at[pl.ds(0, BG)], gs_ref.at[pl.ds(cb * BG, BG)])

    return k(h_pad, idx_d2, idx_s2)


def _sc_scatter(msg, idx_d2, zeros_np):
    @functools.partial(
        pl.kernel,
        mesh=_mesh(),
        out_type=jax.ShapeDtypeStruct((NC, NP, C), jnp.float32),
        scratch_types=[pltpu.VMEM((CPT, BG), jnp.int32),
                       pltpu.VMEM((BG, C), jnp.float32),
                       pltpu.VMEM((BG, C), jnp.float32),
                       pltpu.VMEM_SHARED((NP, C), jnp.float32),
                       pltpu.SemaphoreType.DMA,
                       pltpu.SemaphoreType.DMA],
    )
    def k(m_ref, id_ref, z_ref, out_ref, iv, mva, mvb, acc, sa, sb):
        cid = lax.axis_index("c")
        sid = lax.axis_index("s")
        wid = sid * NC + cid
        rpt = NP // NS
        pltpu.sync_copy(z_ref.at[pl.ds(sid * rpt, rpt)],
                        acc.at[pl.ds(sid * rpt, rpt)])
        cb0 = wid * CPT
        pltpu.sync_copy(id_ref.at[pl.ds(cb0, CPT)], iv)
        plsc.subcore_barrier()
        pltpu.async_copy(m_ref.at[pl.ds(cb0 * BG, BG)], mva, sa)

        def group(g, _):
            cb = cb0 + g

            @pl.when(g % 2 == 0)
            def _():
                @pl.when(g + 1 < CPT)
                def _():
                    pltpu.async_copy(m_ref.at[pl.ds((cb + 1) * BG, BG)], mvb, sb)
                pltpu.make_async_copy(m_ref.at[pl.ds(cb * BG, BG)],
                                      mva, sa).wait()
                pltpu.sync_copy(mva, acc.at[iv.at[g]], add=True)

            @pl.when(g % 2 == 1)
            def _():
                @pl.when(g + 1 < CPT)
                def _():
                    pltpu.async_copy(m_ref.at[pl.ds((cb + 1) * BG, BG)], mva, sa)
                pltpu.make_async_copy(m_ref.at[pl.ds(cb * BG, BG)],
                                      mvb, sb).wait()
                pltpu.sync_copy(mvb, acc.at[iv.at[g]], add=True)

            return 0

        lax.fori_loop(0, CPT, group, 0)
        plsc.subcore_barrier()
        pltpu.sync_copy(acc.at[pl.ds(sid * rpt, rpt)],
                        out_ref.at[cid, pl.ds(sid * rpt, rpt)])

    return k(msg, idx_d2, zeros_np)


def _sc_counts(idx_d2, zeros_np, ones_e):
    cp = _sc_scatter(ones_e, idx_d2, zeros_np)
    return (cp[0] + cp[1])[:, :16]


def _k_pre(x0_ref, x1_ref, am_ref, wa_ref, wv_ref, ba_ref, bv_ref,
           y_ref, st_ref, acc):
    i = pl.program_id(0)
    ya = jnp.dot(x0_ref[...], wa_ref[...], preferred_element_type=jnp.float32) + ba_ref[...]
    yv = jnp.dot(x1_ref[...], wv_ref[...], preferred_element_type=jnp.float32) + bv_ref[...]
    m = jnp.broadcast_to(am_ref[...][:, 0:1], (BN, C))
    y = m * ya + (1.0 - m) * yv
    y_ref[...] = y
    row = i * BN + lax.broadcasted_iota(jnp.int32, (BN, C), 0)
    rm = jnp.where(row < N, 1.0, 0.0)

    @pl.when(i == 0)
    def _():
        acc[...] = jnp.zeros_like(acc)

    ym = y * rm
    acc[0:1, :] += jnp.sum(ym, axis=0, keepdims=True)
    acc[1:2, :] += jnp.sum(ym * y, axis=0, keepdims=True)

    @pl.when(i == pl.num_programs(0) - 1)
    def _():
        st_ref[...] = acc[...]


def _tc_pre(x0p, x1p, am8, W0a, W0v, b0a, b0v):
    grid = NP // BN
    return pl.pallas_call(
        _k_pre,
        grid=(grid,),
        in_specs=[pl.BlockSpec((BN, C), lambda i: (i, 0)),
                  pl.BlockSpec((BN, C), lambda i: (i, 0)),
                  pl.BlockSpec((BN, 8), lambda i: (i, 0)),
                  pl.BlockSpec((C, C), lambda i: (0, 0)),
                  pl.BlockSpec((C, C), lambda i: (0, 0)),
                  pl.BlockSpec((1, C), lambda i: (0, 0)),
                  pl.BlockSpec((1, C), lambda i: (0, 0))],
        out_specs=[pl.BlockSpec((BN, C), lambda i: (i, 0)),
                   pl.BlockSpec((8, C), lambda i: (0, 0))],
        out_shape=[jax.ShapeDtypeStruct((NP, C), jnp.float32),
                   jax.ShapeDtypeStruct((8, C), jnp.float32)],
        scratch_shapes=[pltpu.VMEM((8, C), jnp.float32)],
    )(x0p, x1p, am8, W0a, W0v, b0a, b0v)


def _k_statsA(gd_ref, gs_ref, st_ref, acc):
    i = pl.program_id(0)

    @pl.when(i == 0)
    def _():
        acc[...] = jnp.zeros_like(acc)

    xi = gd_ref[...]
    xd = gs_ref[...] - xi
    acc[0:1, :] += jnp.sum(xi, axis=0, keepdims=True)
    acc[1:2, :] += jnp.sum(xi * xi, axis=0, keepdims=True)
    acc[2:3, :] += jnp.sum(xd, axis=0, keepdims=True)
    acc[3:4, :] += jnp.sum(xd * xd, axis=0, keepdims=True)

    @pl.when(i == pl.num_programs(0) - 1)
    def _():
        st_ref[...] = acc[...]


def _tc_statsA(gd, gs):
    return pl.pallas_call(
        _k_statsA,
        grid=(EPAD // BE,),
        in_specs=[pl.BlockSpec((BE, C), lambda i: (i, 0)),
                  pl.BlockSpec((BE, C), lambda i: (i, 0))],
        out_specs=pl.BlockSpec((8, C), lambda i: (0, 0)),
        out_shape=jax.ShapeDtypeStruct((8, C), jnp.float32),
        scratch_shapes=[pltpu.VMEM((8, C), jnp.float32)],
    )(gd, gs)


def _k_passB(gd_ref, gs_ref, w8_ref, sa_ref, ha_ref, sb_ref, hb_ref,
             w1a_ref, w1b_ref, h1_ref, st_ref, acc):
    i = pl.program_id(0)

    @pl.when(i == 0)
    def _():
        acc[...] = jnp.zeros_like(acc)

    xi = gd_ref[...]
    xd = gs_ref[...] - xi
    ra = jnp.maximum(xi * sa_ref[...] + ha_ref[...], 0.0)
    rb = jnp.maximum(xd * sb_ref[...] + hb_ref[...], 0.0)
    h1 = (jnp.dot(ra, w1a_ref[...], preferred_element_type=jnp.float32)
          + jnp.dot(rb, w1b_ref[...], preferred_element_type=jnp.float32))
    h1_ref[...] = h1
    m = jnp.broadcast_to(w8_ref[...][:, 0:1], (BE, C))
    hm = h1 * m
    acc[0:1, :] += jnp.sum(hm, axis=0, keepdims=True)
    acc[1:2, :] += jnp.sum(hm * h1, axis=0, keepdims=True)

    @pl.when(i == pl.num_programs(0) - 1)
    def _():
        st_ref[...] = acc[...]


def _tc_passB(gd, gs, w8, sca, sha, scb, shb, W1a, W1b):
    return pl.pallas_call(
        _k_passB,
        grid=(EPAD // BE,),
        in_specs=[pl.BlockSpec((BE, C), lambda i: (i, 0)),
                  pl.BlockSpec((BE, C), lambda i: (i, 0)),
                  pl.BlockSpec((BE, 8), lambda i: (i, 0)),
                  pl.BlockSpec((1, C), lambda i: (0, 0)),
                  pl.BlockSpec((1, C), lambda i: (0, 0)),
                  pl.BlockSpec((1, C), lambda i: (0, 0)),
                  pl.BlockSpec((1, C), lambda i: (0, 0)),
                  pl.BlockSpec((C, C), lambda i: (0, 0)),
                  pl.BlockSpec((C, C), lambda i: (0, 0))],
        out_specs=[pl.BlockSpec((BE, C), lambda i: (i, 0)),
                   pl.BlockSpec((8, C), lambda i: (0, 0))],
        out_shape=[jax.ShapeDtypeStruct((EPAD, C), jnp.float32),
                   jax.ShapeDtypeStruct((8, C), jnp.float32)],
        scratch_shapes=[pltpu.VMEM((8, C), jnp.float32)],
    )(gd, gs, w8, sca, sha, scb, shb, W1a, W1b)


def _k_passC(h1_ref, sc_ref, sh_ref, w2_ref, msg_ref):
    r = jnp.maximum(h1_ref[...] * sc_ref[...] + sh_ref[...], 0.0)
    msg_ref[...] = jnp.dot(r, w2_ref[...], preferred_element_type=jnp.float32)


def _tc_passC(h1, sc2, sh2, W2):
    return pl.pallas_call(
        _k_passC,
        grid=(EPAD // BE,),
        in_specs=[pl.BlockSpec((BE, C), lambda i: (i, 0)),
                  pl.BlockSpec((1, C), lambda i: (0, 0)),
                  pl.BlockSpec((1, C), lambda i: (0, 0)),
                  pl.BlockSpec((C, C), lambda i: (0, 0))],
        out_specs=pl.BlockSpec((BE, C), lambda i: (i, 0)),
        out_shape=jax.ShapeDtypeStruct((EPAD, C), jnp.float32),
    )(h1, sc2, sh2, W2)


def _k_combine(a0_ref, a1_ref, ct_ref, res_ref, y_ref, st_ref, acc):
    i = pl.program_id(0)
    c = jnp.broadcast_to(ct_ref[...][:, 0:1], (BN, C))
    inv = 1.0 / jnp.maximum(c, 1.0)
    row = i * BN + lax.broadcasted_iota(jnp.int32, (BN, C), 0)
    rm = jnp.where(row < N, 1.0, 0.0)
    y = ((a0_ref[0] + a1_ref[0]) * inv + res_ref[...]) * rm
    y_ref[...] = y

    @pl.when(i == 0)
    def _():
        acc[...] = jnp.zeros_like(acc)

    acc[0:1, :] += jnp.sum(y, axis=0, keepdims=True)
    acc[1:2, :] += jnp.sum(y * y, axis=0, keepdims=True)

    @pl.when(i == pl.num_programs(0) - 1)
    def _():
        st_ref[...] = acc[...]


def _tc_combine(acc2, cnt16, res):
    return pl.pallas_call(
        _k_combine,
        grid=(NP // BN,),
        in_specs=[pl.BlockSpec((1, BN, C), lambda i: (0, i, 0)),
                  pl.BlockSpec((1, BN, C), lambda i: (1, i, 0)),
                  pl.BlockSpec((BN, 16), lambda i: (i, 0)),
                  pl.BlockSpec((BN, C), lambda i: (i, 0))],
        out_specs=[pl.BlockSpec((BN, C), lambda i: (i, 0)),
                   pl.BlockSpec((8, C), lambda i: (0, 0))],
        out_shape=[jax.ShapeDtypeStruct((NP, C), jnp.float32),
                   jax.ShapeDtypeStruct((8, C), jnp.float32)],
        scratch_shapes=[pltpu.VMEM((8, C), jnp.float32)],
    )(acc2, acc2, cnt16, res)


def _k_norm(y_ref, sc_ref, sh_ref, h_ref):
    i = pl.program_id(0)
    row = i * BN + lax.broadcasted_iota(jnp.int32, (BN, C), 0)
    rm = jnp.where(row < N, 1.0, 0.0)
    h_ref[...] = jnp.maximum(y_ref[...] * sc_ref[...] + sh_ref[...], 0.0) * rm


def _tc_norm(y, sc, sh):
    return pl.pallas_call(
        _k_norm,
        grid=(NP // BN,),
        in_specs=[pl.BlockSpec((BN, C), lambda i: (i, 0)),
                  pl.BlockSpec((1, C), lambda i: (0, 0)),
                  pl.BlockSpec((1, C), lambda i: (0, 0))],
        out_specs=pl.BlockSpec((BN, C), lambda i: (i, 0)),
        out_shape=jax.ShapeDtypeStruct((NP, C), jnp.float32),
    )(y, sc, sh)


def _k_final(a0_ref, a1_ref, ct_ref, res_ref, fw_ref, fb_ref, o_ref):
    c = jnp.broadcast_to(ct_ref[...][:, 0:1], (1000, C))
    inv = 1.0 / jnp.maximum(c, 1.0)
    y = (a0_ref[0] + a1_ref[0]) * inv + res_ref[...]
    o_ref[...] = jnp.dot(y, fw_ref[...], preferred_element_type=jnp.float32) + fb_ref[...]


def _tc_final(acc2, cnt16, res, fcW, fcb):
    return pl.pallas_call(
        _k_final,
        grid=(10,),
        in_specs=[pl.BlockSpec((1, 1000, C), lambda i: (0, i, 0)),
                  pl.BlockSpec((1, 1000, C), lambda i: (1, i, 0)),
                  pl.BlockSpec((1000, 16), lambda i: (i, 0)),
                  pl.BlockSpec((1000, C), lambda i: (i, 0)),
                  pl.BlockSpec((C, 2), lambda i: (0, 0)),
                  pl.BlockSpec((1, 2), lambda i: (0, 0))],
        out_specs=pl.BlockSpec((1000, 2), lambda i: (i, 0)),
        out_shape=jax.ShapeDtypeStruct((N, 2), jnp.float32),
    )(acc2, acc2, cnt16, res, fcW, fcb)


def _fold(s_sum, s_sq, cnt, g, b):
    m = s_sum / cnt
    v = s_sq / cnt - m * m
    rstd = g / jnp.sqrt(v + EPS)
    return rstd.reshape(1, C), (b - m * rstd).reshape(1, C)


def _conv(h, idx_s, idx_d, w8, cnt, bn1g, bn1b, W1, bn2g, bn2b, W2, zeros_np):
    gd, gs = _sc_gather(h, idx_d, idx_s)
    st1 = _tc_statsA(gd, gs)
    sca, sha = _fold(st1[0], st1[1], cnt, bn1g[:C], bn1b[:C])
    scb, shb = _fold(st1[2], st1[3], cnt, bn1g[C:], bn1b[C:])
    h1, st2 = _tc_passB(gd, gs, w8, sca, sha, scb, shb, W1[:C], W1[C:])
    sc2, sh2 = _fold(st2[0], st2[1], cnt, bn2g, bn2b)
    msg = _tc_passC(h1, sc2, sh2, W2)
    return _sc_scatter(msg, idx_d, zeros_np)


def kernel(x, edge_index, edge_delta, edge_self, audio_node_mask,
           W0a, b0a, W0v, b0v, g0, be0,
           l1_bn1g, l1_bn1b, l1_W1, l1_bn2g, l1_bn2b, l1_W2,
           l2_bn1g, l2_bn1b, l2_W1, l2_bn2g, l2_bn2b, l2_W2,
           l3_bn1g, l3_bn1b, l3_W1, l3_bn2g, l3_bn2b, l3_W2,
           l4_bn1g, l4_bn1b, l4_W1, l4_bn2g, l4_bn2b, l4_W2,
           g1, be1, g2, be2, g3, be3, fcW, fcb):
    f32 = jnp.float32
    src = edge_index[0]
    dst = edge_index[1]
    m1 = edge_delta < 1
    m2 = ((edge_delta >= 1) & (edge_delta < 3)) | (edge_self == 1)

    spread = N + (jnp.arange(EPAD, dtype=jnp.int32) % (NP - N))
    srcp = jnp.pad(src, (0, EPAD - E))
    dstp = jnp.pad(dst, (0, EPAD - E))

    def prep(m):
        w = m.astype(f32)
        mp = jnp.pad(m, (0, EPAD - E))
        s = jnp.where(mp, srcp, spread).astype(jnp.int32).reshape(NCHUNK, BG)
        d = jnp.where(mp, dstp, spread).astype(jnp.int32).reshape(NCHUNK, BG)
        w8 = jnp.pad(jnp.tile(w[:, None], (1, 8)), [(0, EPAD - E), (0, 0)])
        return s, d, w8

    s1i, d1i, w8_1 = prep(m1)
    s2i, d2i, w8_2 = prep(m2)

    zeros_np = jnp.zeros((NP, C), f32)
    ones_e = jnp.ones((EPAD, C), f32)

    cnt16_1 = _sc_counts(d1i, zeros_np, ones_e)
    cnt16_2 = _sc_counts(d2i, zeros_np, ones_e)
    cnt1 = jnp.sum(cnt16_1[:N, 0])
    cnt2 = jnp.sum(cnt16_2[:N, 0])
    cnt1 = jnp.maximum(cnt1, 1.0)
    cnt2 = jnp.maximum(cnt2, 1.0)

    pad_n = [(0, NP - N), (0, 0)]
    x0p = jnp.pad(x[:, 0, :], pad_n)
    x1p = jnp.pad(x[:, 1, :], pad_n)
    am8 = jnp.pad(jnp.tile(audio_node_mask.astype(f32)[:, None], (1, 8)),
                  [(0, NP - N), (0, 0)])

    y0, st0 = _tc_pre(x0p, x1p, am8, W0a, W0v,
                      b0a.reshape(1, C), b0v.reshape(1, C))
    sc0, sh0 = _fold(st0[0], st0[1], float(N), g0, be0)
    gf = _tc_norm(y0, sc0, sh0)

    params = [(l1_bn1g, l1_bn1b, l1_W1, l1_bn2g, l1_bn2b, l1_W2),
              (l2_bn1g, l2_bn1b, l2_W1, l2_bn2g, l2_bn2b, l2_W2),
              (l3_bn1g, l3_bn1b, l3_W1, l3_bn2g, l3_bn2b, l3_W2),
              (l4_bn1g, l4_bn1b, l4_W1, l4_bn2g, l4_bn2b, l4_W2)]
    gbs = [(g1, be1), (g2, be2), (g3, be3)]

    h = gf
    for l in range(4):
        p = params[l]
        acc_a = _conv(h, s1i, d1i, w8_1, cnt1, *p, zeros_np)
        a, _ = _tc_combine(acc_a, cnt16_1, zeros_np)
        acc_b = _conv(a, s2i, d2i, w8_2, cnt2, *p, zeros_np)
        res = zeros_np if l == 0 else h
        if l < 3:
            y, stn = _tc_combine(acc_b, cnt16_2, res)
            scn, shn = _fold(stn[0], stn[1], float(N), gbs[l][0], gbs[l][1])
            h = _tc_norm(y, scn, shn)
        else:
            return _tc_final(acc_b, cnt16_2, res, fcW, fcb.reshape(1, 2))
